# Initial kernel scaffold; baseline (speedup 1.0000x reference)
import jax
import jax.numpy as jnp
from jax.experimental import pallas as pl


def kernel(h, policies, actions, obs_proc, W_fc, W_attn):
    raise NotImplementedError("write your pallas kernel here")


# speedup vs baseline: 1.4903x; 1.4903x over previous
import functools

import jax
import jax.numpy as jnp
from jax.experimental import pallas as pl

A = 32
ACT = 16
IN_DIM = 128
OUT_DIM = 128
D_OBS = 128
B = 256


def _body(h_ref, pi_ref, act_ref, obs_ref, wfc_ref, wattn_ref,
          out_ref, wout_ref, *, E):
    z = jax.lax.dot_general(
        h_ref[...], wfc_ref[...],
        dimension_numbers=(((1,), (1,)), ((), ())),
        preferred_element_type=jnp.float32)
    attn = wattn_ref[...].reshape(2, OUT_DIM)
    sd = jax.lax.dot_general(
        z, attn, dimension_numbers=(((1,), (1,)), ((), ())),
        preferred_element_type=jnp.float32)
    sv = sd[:, 0].reshape(E, A)
    dv = sd[:, 1].reshape(E, A)
    e = sv[:, None, :] + dv[:, :, None]
    e = jnp.where(e >= 0.0, e, 0.01 * e)
    w = jax.nn.sigmoid(e)

    pi = pi_ref[...]
    q = act_ref[...] - pi
    P = jnp.sum(pi, axis=1)
    r = jnp.stack([
        jnp.dot(w[b], q[b], preferred_element_type=jnp.float32)
        for b in range(E)], axis=0)

    inv_a = jnp.float32(1.0 / A)
    zm = (P[:, None, None, :] + r[:, :, None, :]
          - w[..., None] * q[:, None, :, :]) * inv_a

    obs = obs_ref[...].reshape(E, 1, A, D_OBS)
    out_ref[:, :, :D_OBS] = jnp.broadcast_to(
        obs, (E, A, A, D_OBS)).reshape(E * A, A, D_OBS)
    out_ref[:, :, D_OBS:] = zm.reshape(E * A, A, ACT)
    wout_ref[...] = w.reshape(E * A, A, 1)


@jax.jit
def kernel(h, policies, actions, obs_proc, W_fc, W_attn):
    E = 8
    grid = (B // E,)
    out_shapes = (
        jax.ShapeDtypeStruct((B * A, A, D_OBS + ACT), jnp.float32),
        jax.ShapeDtypeStruct((B * A, A, 1), jnp.float32),
    )
    return pl.pallas_call(
        functools.partial(_body, E=E),
        grid=grid,
        in_specs=[
            pl.BlockSpec((E * A, IN_DIM), lambda b: (b, 0)),
            pl.BlockSpec((E, A, ACT), lambda b: (b, 0, 0)),
            pl.BlockSpec((E, A, ACT), lambda b: (b, 0, 0)),
            pl.BlockSpec((E * A, D_OBS), lambda b: (b, 0)),
            pl.BlockSpec((OUT_DIM, IN_DIM), lambda b: (0, 0)),
            pl.BlockSpec((1, 2 * OUT_DIM), lambda b: (0, 0)),
        ],
        out_specs=(
            pl.BlockSpec((E * A, A, D_OBS + ACT), lambda b: (b, 0, 0)),
            pl.BlockSpec((E * A, A, 1), lambda b: (b, 0, 0)),
        ),
        out_shape=out_shapes,
    )(h, policies, actions, obs_proc, W_fc, W_attn)


# speedup vs baseline: 1.4924x; 1.0014x over previous
import functools

import jax
import jax.numpy as jnp
from jax.experimental import pallas as pl

A = 32
ACT = 16
IN_DIM = 128
OUT_DIM = 128
D_OBS = 128
B = 256


def _body(h_ref, pi_ref, act_ref, obs_ref, wfc_ref, wattn_ref,
          out_ref, wout_ref, *, E):
    z = jax.lax.dot_general(
        h_ref[...], wfc_ref[...],
        dimension_numbers=(((1,), (1,)), ((), ())),
        preferred_element_type=jnp.float32)
    attn = wattn_ref[...].reshape(2, OUT_DIM)
    sd = jax.lax.dot_general(
        z, attn, dimension_numbers=(((1,), (1,)), ((), ())),
        preferred_element_type=jnp.float32)
    sv = sd[:, 0].reshape(E, A)
    dv = sd[:, 1].reshape(E, A)
    e = sv[:, None, :] + dv[:, :, None]
    e = jnp.where(e >= 0.0, e, 0.01 * e)
    w = jax.nn.sigmoid(e)

    pi = pi_ref[...]
    q = act_ref[...] - pi
    P = jnp.sum(pi, axis=1)
    t = w[..., None] * q[:, None, :, :]
    r = jnp.sum(t, axis=2)

    inv_a = jnp.float32(1.0 / A)
    zm = (P[:, None, None, :] + r[:, :, None, :] - t) * inv_a

    obs = obs_ref[...].reshape(E, 1, A, D_OBS)
    out_ref[:, :, :D_OBS] = jnp.broadcast_to(
        obs, (E, A, A, D_OBS)).reshape(E * A, A, D_OBS)
    out_ref[:, :, D_OBS:] = zm.reshape(E * A, A, ACT)
    wout_ref[...] = w.reshape(E * A, A, 1)


@jax.jit
def kernel(h, policies, actions, obs_proc, W_fc, W_attn):
    E = 8
    grid = (B // E,)
    out_shapes = (
        jax.ShapeDtypeStruct((B * A, A, D_OBS + ACT), jnp.float32),
        jax.ShapeDtypeStruct((B * A, A, 1), jnp.float32),
    )
    return pl.pallas_call(
        functools.partial(_body, E=E),
        grid=grid,
        in_specs=[
            pl.BlockSpec((E * A, IN_DIM), lambda b: (b, 0)),
            pl.BlockSpec((E, A, ACT), lambda b: (b, 0, 0)),
            pl.BlockSpec((E, A, ACT), lambda b: (b, 0, 0)),
            pl.BlockSpec((E * A, D_OBS), lambda b: (b, 0)),
            pl.BlockSpec((OUT_DIM, IN_DIM), lambda b: (0, 0)),
            pl.BlockSpec((1, 2 * OUT_DIM), lambda b: (0, 0)),
        ],
        out_specs=(
            pl.BlockSpec((E * A, A, D_OBS + ACT), lambda b: (b, 0, 0)),
            pl.BlockSpec((E * A, A, 1), lambda b: (b, 0, 0)),
        ),
        out_shape=out_shapes,
    )(h, policies, actions, obs_proc, W_fc, W_attn)
